# baseline (speedup 1.0000x reference)
import functools

import jax
import jax.numpy as jnp
from jax import lax
from jax.experimental import pallas as pl
from jax.experimental.pallas import tpu as pltpu
from jax.experimental.pallas import tpu_sc as plsc

B, S, H = 1, 2048, 2048
NH, NKV, HD = 16, 8, 128
E, TOPK, FF = 8, 2, 2048
EPS = 1e-5
ATTN_CAP = 30.0
ROUTER_CAP = 30.0
THETA = 10000.0
ATTN_MULT = 1.0

TM = 128
G = S * TOPK // TM + E
XS = G * TM
FN = 512
RT = 256


def _rms(x, w):
    v = jnp.mean(jnp.square(x), axis=-1, keepdims=True)
    return (x * lax.rsqrt(v + EPS)) * w


def _rope(x, pos):
    inv = 1.0 / (THETA ** (jnp.arange(0, HD, 2, dtype=jnp.float32) / HD))
    f = pos.astype(jnp.float32)[..., None] * inv
    cos = jnp.concatenate([jnp.cos(f), jnp.cos(f)], axis=-1)[:, :, None, :]
    sin = jnp.concatenate([jnp.sin(f), jnp.sin(f)], axis=-1)[:, :, None, :]
    x1, x2 = jnp.split(x, 2, axis=-1)
    rot = jnp.concatenate([-x2, x1], axis=-1)
    return x * cos + rot * sin


def _route_body(ti_ref, tv_ref, p0_ref, p1_ref, w0_ref, w1_ref, te_ref, tot_ref):
    T = S
    A = T * TOPK
    NB = A // TM

    tv = tv_ref[...]
    denom = tv[:, 0:1] + tv[:, 1:2]
    w0_ref[...] = tv[:, 0:1] / denom
    w1_ref[...] = tv[:, 1:2] / denom

    ti = ti_ref[...]
    ei = lax.broadcasted_iota(jnp.int32, (T, E), 1)
    oh1 = (ei == ti[:, 0:1]).astype(jnp.float32)
    oh2 = (ei == ti[:, 1:2]).astype(jnp.float32)

    M = jnp.concatenate([oh1, oh2], axis=0)

    ri = lax.broadcasted_iota(jnp.int32, (TM, TM), 0)
    ci = lax.broadcasted_iota(jnp.int32, (TM, TM), 1)
    Linc = (ri >= ci).astype(jnp.float32)
    hp = lax.Precision.HIGHEST
    Cb = [jnp.dot(Linc, M[b * TM : (b + 1) * TM, :], precision=hp,
                  preferred_element_type=jnp.float32) for b in range(NB)]
    C = jnp.concatenate(Cb, axis=0)
    Ssum = jnp.concatenate([c[TM - 1 : TM, :] for c in Cb], axis=0)
    rb = lax.broadcasted_iota(jnp.int32, (NB, NB), 0)
    cb = lax.broadcasted_iota(jnp.int32, (NB, NB), 1)
    Lstr = (rb > cb).astype(jnp.float32)
    P = jnp.dot(Lstr, Ssum, precision=hp, preferred_element_type=jnp.float32)
    blk = lax.broadcasted_iota(jnp.int32, (A, 1), 0) // TM
    Rep = (blk == lax.broadcasted_iota(jnp.int32, (A, NB), 1)).astype(jnp.float32)
    Pbig = jnp.dot(Rep, P, precision=hp, preferred_element_type=jnp.float32)
    rank = jnp.sum((C + Pbig - 1.0) * M, axis=1, keepdims=True)

    counts = Ssum[NB - 1 : NB, :] + P[NB - 1 : NB, :]
    ci32 = counts.astype(jnp.int32)
    tiles = (ci32 + (TM - 1)) >> 7
    re8 = lax.broadcasted_iota(jnp.int32, (E, E), 0)
    ce8 = lax.broadcasted_iota(jnp.int32, (E, E), 1)
    U8 = (re8 < ce8).astype(jnp.float32)
    toff = jnp.dot(tiles.astype(jnp.float32), U8, precision=hp,
                   preferred_element_type=jnp.float32)
    padded_off = toff * float(TM)
    offa = jnp.sum(M * padded_off, axis=1, keepdims=True)
    pos = (rank + offa).astype(jnp.int32)
    p0_ref[...] = pos[:T]
    p1_ref[...] = pos[T:]

    tei = lax.broadcasted_iota(jnp.int32, (TM, E), 0)
    cnt = jnp.sum((tei >= toff.astype(jnp.int32)).astype(jnp.int32),
                  axis=1, keepdims=True)
    te_ref[...] = jnp.maximum(cnt - 1, 0)
    tot_ref[...] = jnp.sum(tiles, axis=1, keepdims=True)


def _route(topi, topv):
    return pl.pallas_call(
        _route_body,
        out_shape=[
            jax.ShapeDtypeStruct((S, 1), jnp.int32),
            jax.ShapeDtypeStruct((S, 1), jnp.int32),
            jax.ShapeDtypeStruct((S, 1), jnp.float32),
            jax.ShapeDtypeStruct((S, 1), jnp.float32),
            jax.ShapeDtypeStruct((TM, 1), jnp.int32),
            jax.ShapeDtypeStruct((1, 1), jnp.int32),
        ],
    )(topi, topv)


def _sc_info():
    info = plsc.get_sparse_core_info()
    return info.num_cores, info.num_subcores


def _dispatch_sc(xn, p0, p1):
    NC, NS = _sc_info()
    NW = NC * NS
    tpw = S // NW
    CH = 16
    mesh = plsc.VectorSubcoreMesh(core_axis_name="c", subcore_axis_name="s")

    @functools.partial(
        pl.kernel, mesh=mesh,
        out_type=jax.ShapeDtypeStruct((XS, H), jnp.float32),
        scratch_types=[
            pltpu.VMEM((CH,), jnp.int32),
            pltpu.VMEM((CH,), jnp.int32),
            pltpu.VMEM((CH, H), jnp.float32),
            pltpu.SemaphoreType.DMA,
        ],
    )
    def k(x_hbm, p0_hbm, p1_hbm, xs_hbm, i0_v, i1_v, rows_v, sem):
        wid = lax.axis_index("s") * NC + lax.axis_index("c")
        base0 = wid * tpw
        for c in range(tpw // CH):
            base = base0 + c * CH
            pltpu.sync_copy(p0_hbm.at[pl.ds(base, CH)], i0_v)
            pltpu.sync_copy(p1_hbm.at[pl.ds(base, CH)], i1_v)
            pltpu.sync_copy(x_hbm.at[pl.ds(base, CH)], rows_v)
            pltpu.async_copy(rows_v, xs_hbm.at[i0_v], sem).wait()
            pltpu.async_copy(rows_v, xs_hbm.at[i1_v], sem).wait()

    return k(xn, p0, p1)


def _combine_sc(ys, p0, p1):
    NC, NS = _sc_info()
    NW = NC * NS
    tpw = S // NW
    CH = 16
    mesh = plsc.VectorSubcoreMesh(core_axis_name="c", subcore_axis_name="s")

    @functools.partial(
        pl.kernel, mesh=mesh,
        out_type=jax.ShapeDtypeStruct((TOPK * S, H), jnp.float32),
        scratch_types=[
            pltpu.VMEM((CH,), jnp.int32),
            pltpu.VMEM((CH, H), jnp.float32),
            pltpu.SemaphoreType.DMA,
        ],
    )
    def k(ys_hbm, p0_hbm, p1_hbm, yp_hbm, idx_v, rows_v, sem):
        wid = lax.axis_index("s") * NC + lax.axis_index("c")
        base0 = wid * tpw
        for kk, p_hbm in ((0, p0_hbm), (1, p1_hbm)):
            for c in range(tpw // CH):
                base = base0 + c * CH
                pltpu.sync_copy(p_hbm.at[pl.ds(base, CH)], idx_v)
                pltpu.async_copy(ys_hbm.at[idx_v], rows_v, sem).wait()
                pltpu.sync_copy(rows_v, yp_hbm.at[pl.ds(kk * S + base, CH)])

    return k(ys, p0, p1)


def _gelu(x):
    return 0.5 * x * (1.0 + lax.erf(x * 0.7071067811865475))


def _gmm1_body(te_ref, tot_ref, xs_ref, wg_ref, wu_ref, o_ref):
    m = pl.program_id(1)

    @pl.when(m < tot_ref[0])
    def _():
        x = xs_ref[...].astype(jnp.bfloat16)
        g = jnp.dot(x, wg_ref[0], preferred_element_type=jnp.float32)
        u = jnp.dot(x, wu_ref[0], preferred_element_type=jnp.float32)
        o_ref[...] = (_gelu(g) * u).astype(jnp.bfloat16)


def _gmm1(te, tot, xs, w_gate_up):
    grid_spec = pltpu.PrefetchScalarGridSpec(
        num_scalar_prefetch=2,
        grid=(FF // FN, G),
        in_specs=[
            pl.BlockSpec((TM, H), lambda n, m, te, tot: (m, 0)),
            pl.BlockSpec((1, H, FN), lambda n, m, te, tot: (te[m], 0, n)),
            pl.BlockSpec((1, H, FN), lambda n, m, te, tot: (te[m], 0, n + FF // FN)),
        ],
        out_specs=pl.BlockSpec((TM, FN), lambda n, m, te, tot: (m, n)),
    )
    return pl.pallas_call(
        _gmm1_body,
        grid_spec=grid_spec,
        out_shape=jax.ShapeDtypeStruct((XS, FF), jnp.bfloat16),
    )(te, tot, xs, w_gate_up, w_gate_up)


def _gmm2_body(te_ref, tot_ref, a_ref, wd_ref, o_ref):
    m = pl.program_id(1)

    @pl.when(m < tot_ref[0])
    def _():
        o_ref[...] = jnp.dot(a_ref[...], wd_ref[0],
                             preferred_element_type=jnp.float32)


def _gmm2(te, tot, act, w_down):
    grid_spec = pltpu.PrefetchScalarGridSpec(
        num_scalar_prefetch=2,
        grid=(H // FN, G),
        in_specs=[
            pl.BlockSpec((TM, FF), lambda n, m, te, tot: (m, 0)),
            pl.BlockSpec((1, FF, FN), lambda n, m, te, tot: (te[m], 0, n)),
        ],
        out_specs=pl.BlockSpec((TM, FN), lambda n, m, te, tot: (m, n)),
    )
    return pl.pallas_call(
        _gmm2_body,
        grid_spec=grid_spec,
        out_shape=jax.ShapeDtypeStruct((XS, H), jnp.float32),
    )(te, tot, act, w_down)


def _fin_body(y0_ref, y1_ref, w0_ref, w1_ref, hid_ref, qm_ref, o_ref):
    moe = y0_ref[...] * w0_ref[...] + y1_ref[...] * w1_ref[...]
    o_ref[...] = hid_ref[...] + _rms(moe, qm_ref[...])


def _finalize(yp, w0, w1, hidden, qm):
    return pl.pallas_call(
        _fin_body,
        grid=(S // RT,),
        in_specs=[
            pl.BlockSpec((RT, H), lambda r: (r, 0)),
            pl.BlockSpec((RT, H), lambda r: (r + S // RT, 0)),
            pl.BlockSpec((RT, 1), lambda r: (r, 0)),
            pl.BlockSpec((RT, 1), lambda r: (r, 0)),
            pl.BlockSpec((RT, H), lambda r: (r, 0)),
            pl.BlockSpec((1, H), lambda r: (0, 0)),
        ],
        out_specs=pl.BlockSpec((RT, H), lambda r: (r, 0)),
        out_shape=jax.ShapeDtypeStruct((S, H), jnp.float32),
    )(yp, yp, w0, w1, hidden, qm)


def kernel(positions, hidden_states, w_qkv, w_o, gate_w, w_gate_up, w_down,
           pre_attn_norm_w, post_attn_norm_w, pre_moe_norm_w, post_moe_norm_w):
    h = _rms(hidden_states, pre_attn_norm_w)
    qkv = h @ w_qkv
    q = qkv[..., : NH * HD].reshape(B, S, NH, HD)
    k = qkv[..., NH * HD : NH * HD + NKV * HD].reshape(B, S, NKV, HD)
    v = qkv[..., NH * HD + NKV * HD :].reshape(B, S, NKV, HD)
    q = _rope(q, positions)
    k = _rope(k, positions)
    k = jnp.repeat(k, NH // NKV, axis=2)
    v = jnp.repeat(v, NH // NKV, axis=2)
    scores = jnp.einsum('bqhd,bkhd->bhqk', q, k) * (HD ** -0.5)
    scores = ATTN_CAP * jnp.tanh(scores / ATTN_CAP)
    mask = jnp.tril(jnp.ones((S, S), dtype=bool))
    scores = jnp.where(mask[None, None], scores, -1e9)
    p = jax.nn.softmax(scores, axis=-1)
    ao = jnp.einsum('bhqk,bkhd->bqhd', p, v).reshape(B, S, NH * HD)
    ao = (ao @ w_o) * ATTN_MULT
    ao = _rms(ao, post_attn_norm_w)
    hidden = hidden_states + ao
    x = _rms(hidden, pre_moe_norm_w).reshape(-1, H)
    logits = x @ gate_w
    logits = ROUTER_CAP * jnp.tanh(logits / ROUTER_CAP)
    probs = jax.nn.softmax(logits, axis=-1)
    topv, topi = jax.lax.top_k(probs, TOPK)

    p0, p1, w0, w1, te, tot = _route(topi, topv)
    p0f = p0.reshape(S)
    p1f = p1.reshape(S)
    xs = _dispatch_sc(x, p0f, p1f)
    act = _gmm1(te.reshape(TM), tot.reshape(1), xs,
                w_gate_up.astype(jnp.bfloat16))
    ys = _gmm2(te.reshape(TM), tot.reshape(1), act,
               w_down.astype(jnp.bfloat16))
    yp = _combine_sc(ys, p0f, p1f)
    out = _finalize(yp, w0, w1, hidden.reshape(S, H), post_moe_norm_w.reshape(1, H))
    return out.reshape(B, S, H)
